# baseline (speedup 1.0000x reference)
import functools

import jax
import jax.numpy as jnp
from jax import lax
from jax.experimental import pallas as pl
from jax.experimental.pallas import tpu as pltpu
from jax.experimental.pallas import tpu_sc as plsc

N_NODES = 10000
N_EDGES = 160000
IN_DIM = 256
HID_DIM = 512

HALF = IN_DIM // 2
WCOLS = HALF + 16
NC = 2
NS = 16
CHUNK = 128
KBLK = 8
NBLK = 10
EDGES_PER_TILE = NBLK * KBLK * CHUNK
E_PAD = EDGES_PER_TILE * NS
ZROW = 2 * N_NODES
ROWS_PER_TILE = N_NODES // NS


def _sc_scatter(aug, src4, dst5, zrows, out,
                src_v, dst_v, rows0, rows1, shared, sem0, sem1):
    c = lax.axis_index("c")
    s = lax.axis_index("s")
    pltpu.sync_copy(zrows, shared.at[pl.ds(s * ROWS_PER_TILE, ROWS_PER_TILE)])
    plsc.subcore_barrier()

    rows = (rows0, rows1)
    sems = (sem0, sem1)

    def body(jj, carry):
        pltpu.sync_copy(src4.at[s, jj], src_v)
        pltpu.sync_copy(dst5.at[c, s, jj], dst_v)
        descs = [None, None]
        descs[0] = pltpu.async_copy(aug.at[dst_v.at[0]], rows0, sem0)
        for k in range(KBLK):
            cur = k % 2
            descs[cur].wait()
            if k + 1 < KBLK:
                nxt = (k + 1) % 2
                descs[nxt] = pltpu.async_copy(
                    aug.at[dst_v.at[k + 1]], rows[nxt], sems[nxt])
            pltpu.sync_copy(rows[cur], shared.at[src_v.at[k]], add=True)
        return carry

    lax.fori_loop(0, NBLK, body, 0)
    plsc.subcore_barrier()
    pltpu.sync_copy(
        shared.at[pl.ds(s * ROWS_PER_TILE, ROWS_PER_TILE)],
        out.at[c, pl.ds(s * ROWS_PER_TILE, ROWS_PER_TILE)],
    )


_sc_scatter_call = functools.partial(
    pl.kernel,
    out_type=jax.ShapeDtypeStruct((NC, N_NODES, WCOLS), jnp.float32),
    mesh=plsc.VectorSubcoreMesh(core_axis_name="c", subcore_axis_name="s"),
    scratch_types=[
        pltpu.VMEM((KBLK, CHUNK), jnp.int32),
        pltpu.VMEM((KBLK, CHUNK), jnp.int32),
        pltpu.VMEM((CHUNK, WCOLS), jnp.float32),
        pltpu.VMEM((CHUNK, WCOLS), jnp.float32),
        pltpu.VMEM_SHARED((N_NODES, WCOLS), jnp.float32),
        pltpu.SemaphoreType.DMA,
        pltpu.SemaphoreType.DMA,
    ],
    compiler_params=pltpu.CompilerParams(use_tc_tiling_on_sc=False),
)(_sc_scatter)


def _tc_body(x_ref, n_ref, ws_ref, wn0_ref, wn1_ref, b_ref, o_ref):
    x = x_ref[...]
    nb = n_ref[...]
    sum0 = nb[0, :, :HALF]
    sum1 = nb[1, :, :HALF]
    cnt = nb[0, :, HALF:HALF + 1]
    mask = cnt > 0.0
    safe = jnp.where(mask, cnt, 1.0)
    m0 = jnp.where(mask, sum0 / safe, 0.0)
    m1 = jnp.where(mask, sum1 / safe, 0.0)
    acc = jnp.dot(x, ws_ref[...], preferred_element_type=jnp.float32,
                  precision=lax.Precision.HIGHEST)
    acc += jnp.dot(m0, wn0_ref[...], preferred_element_type=jnp.float32,
                   precision=lax.Precision.HIGHEST)
    acc += jnp.dot(m1, wn1_ref[...], preferred_element_type=jnp.float32,
                   precision=lax.Precision.HIGHEST)
    acc += b_ref[...]
    acc = jnp.maximum(acc, 0.0)
    nrm = jnp.sqrt(jnp.sum(acc * acc, axis=1, keepdims=True)) + 1e-9
    o_ref[...] = acc / nrm


def _tc_call(x, neigh, ws, wn0, wn1, b):
    R = 1000
    grid = (N_NODES // R,)
    return pl.pallas_call(
        _tc_body,
        grid=grid,
        in_specs=[
            pl.BlockSpec((R, IN_DIM), lambda i: (i, 0)),
            pl.BlockSpec((NC, R, WCOLS), lambda i: (0, i, 0)),
            pl.BlockSpec((IN_DIM, HID_DIM), lambda i: (0, 0)),
            pl.BlockSpec((HALF, HID_DIM), lambda i: (0, 0)),
            pl.BlockSpec((HALF, HID_DIM), lambda i: (0, 0)),
            pl.BlockSpec((1, HID_DIM), lambda i: (0, 0)),
        ],
        out_specs=pl.BlockSpec((R, HID_DIM), lambda i: (i, 0)),
        out_shape=jax.ShapeDtypeStruct((N_NODES, HID_DIM), jnp.float32),
    )(x, neigh, ws, wn0, wn1, b)


@jax.jit
def kernel(item_emb, edges, w_self_W, w_self_b, w_neigh_W, w_neigh_b):
    f32 = jnp.float32
    src = edges[:, 0].astype(jnp.int32)
    dst = edges[:, 1].astype(jnp.int32)
    npad = E_PAD - N_EDGES
    srcp = jnp.concatenate([src, jnp.zeros((npad,), jnp.int32)])
    dpad = jnp.full((npad,), ZROW, jnp.int32)
    ones = jnp.ones((N_NODES, 1), f32)
    pad = jnp.zeros((N_NODES, WCOLS - HALF - 1), f32)
    aug = jnp.concatenate([
        jnp.concatenate([item_emb[:, :HALF], ones, pad], axis=1),
        jnp.concatenate([item_emb[:, HALF:], ones, pad], axis=1),
        jnp.zeros((8, WCOLS), f32),
    ], axis=0)
    src4 = srcp.reshape(NS, NBLK, KBLK, CHUNK)
    dst5 = jnp.stack([
        jnp.concatenate([dst, dpad]),
        jnp.concatenate([dst + N_NODES, dpad]),
    ]).reshape(NC, NS, NBLK, KBLK, CHUNK)
    zrows = jnp.zeros((ROWS_PER_TILE, WCOLS), f32)

    neigh = _sc_scatter_call(aug, src4, dst5, zrows)

    bias = (w_self_b + w_neigh_b).reshape(1, HID_DIM)
    return _tc_call(item_emb, neigh, w_self_W,
                    w_neigh_W[:HALF], w_neigh_W[HALF:], bias)


# speedup vs baseline: 1.6303x; 1.6303x over previous
import functools

import jax
import jax.numpy as jnp
from jax import lax
from jax.experimental import pallas as pl
from jax.experimental.pallas import tpu as pltpu
from jax.experimental.pallas import tpu_sc as plsc

N_NODES = 10000
N_EDGES = 160000
IN_DIM = 256
HID_DIM = 512

HALF = IN_DIM // 2
WCOLS = HALF + 16
NC = 2
NS = 16
EDGES_PER_TILE = N_EDGES // NS
CHUNK = 250
NCHUNK = EDGES_PER_TILE // CHUNK
ROWS_PER_TILE = N_NODES // NS


def _sc_scatter(aug, src3, dst4, zrows, out, src_v, dst_v, rows_v, shared, sem):
    c = lax.axis_index("c")
    s = lax.axis_index("s")
    pltpu.sync_copy(zrows, shared.at[pl.ds(s * ROWS_PER_TILE, ROWS_PER_TILE)])
    plsc.subcore_barrier()

    def body(j, carry):
        pltpu.sync_copy(src3.at[s, j], src_v)
        pltpu.sync_copy(dst4.at[c, s, j], dst_v)
        pltpu.async_copy(aug.at[dst_v], rows_v, sem).wait()
        pltpu.sync_copy(rows_v, shared.at[src_v], add=True)
        return carry

    lax.fori_loop(0, NCHUNK, body, 0)
    plsc.subcore_barrier()
    pltpu.sync_copy(
        shared.at[pl.ds(s * ROWS_PER_TILE, ROWS_PER_TILE)],
        out.at[c, pl.ds(s * ROWS_PER_TILE, ROWS_PER_TILE)],
    )


_sc_scatter_call = functools.partial(
    pl.kernel,
    out_type=jax.ShapeDtypeStruct((NC, N_NODES, WCOLS), jnp.float32),
    mesh=plsc.VectorSubcoreMesh(core_axis_name="c", subcore_axis_name="s"),
    scratch_types=[
        pltpu.VMEM((CHUNK,), jnp.int32),
        pltpu.VMEM((CHUNK,), jnp.int32),
        pltpu.VMEM((CHUNK, WCOLS), jnp.float32),
        pltpu.VMEM_SHARED((N_NODES, WCOLS), jnp.float32),
        pltpu.SemaphoreType.DMA,
    ],
    compiler_params=pltpu.CompilerParams(use_tc_tiling_on_sc=False),
)(_sc_scatter)


def _tc_body(x_ref, n_ref, ws_ref, wn0_ref, wn1_ref, b_ref, o_ref):
    x = x_ref[...]
    nb = n_ref[...]
    sum0 = nb[0, :, :HALF]
    sum1 = nb[1, :, :HALF]
    cnt = nb[0, :, HALF:HALF + 1]
    mask = cnt > 0.0
    safe = jnp.where(mask, cnt, 1.0)
    m0 = jnp.where(mask, sum0 / safe, 0.0)
    m1 = jnp.where(mask, sum1 / safe, 0.0)
    acc = jnp.dot(x, ws_ref[...], preferred_element_type=jnp.float32,
                  precision=lax.Precision.HIGHEST)
    acc += jnp.dot(m0, wn0_ref[...], preferred_element_type=jnp.float32,
                   precision=lax.Precision.HIGHEST)
    acc += jnp.dot(m1, wn1_ref[...], preferred_element_type=jnp.float32,
                   precision=lax.Precision.HIGHEST)
    acc += b_ref[...]
    acc = jnp.maximum(acc, 0.0)
    nrm = jnp.sqrt(jnp.sum(acc * acc, axis=1, keepdims=True)) + 1e-9
    o_ref[...] = acc / nrm


def _tc_call(x, neigh, ws, wn0, wn1, b):
    R = 1000
    grid = (N_NODES // R,)
    return pl.pallas_call(
        _tc_body,
        grid=grid,
        in_specs=[
            pl.BlockSpec((R, IN_DIM), lambda i: (i, 0)),
            pl.BlockSpec((NC, R, WCOLS), lambda i: (0, i, 0)),
            pl.BlockSpec((IN_DIM, HID_DIM), lambda i: (0, 0)),
            pl.BlockSpec((HALF, HID_DIM), lambda i: (0, 0)),
            pl.BlockSpec((HALF, HID_DIM), lambda i: (0, 0)),
            pl.BlockSpec((1, HID_DIM), lambda i: (0, 0)),
        ],
        out_specs=pl.BlockSpec((R, HID_DIM), lambda i: (i, 0)),
        out_shape=jax.ShapeDtypeStruct((N_NODES, HID_DIM), jnp.float32),
    )(x, neigh, ws, wn0, wn1, b)


@jax.jit
def kernel(item_emb, edges, w_self_W, w_self_b, w_neigh_W, w_neigh_b):
    f32 = jnp.float32
    src = edges[:, 0].astype(jnp.int32)
    dst = edges[:, 1].astype(jnp.int32)
    ones = jnp.ones((N_NODES, 1), f32)
    pad = jnp.zeros((N_NODES, WCOLS - HALF - 1), f32)
    aug = jnp.concatenate([
        jnp.concatenate([item_emb[:, :HALF], ones, pad], axis=1),
        jnp.concatenate([item_emb[:, HALF:], ones, pad], axis=1),
    ], axis=0)
    src3 = src.reshape(NS, NCHUNK, CHUNK)
    dst4 = jnp.stack([dst, dst + N_NODES]).reshape(NC, NS, NCHUNK, CHUNK)
    zrows = jnp.zeros((ROWS_PER_TILE, WCOLS), f32)

    neigh = _sc_scatter_call(aug, src3, dst4, zrows)

    bias = (w_self_b + w_neigh_b).reshape(1, HID_DIM)
    return _tc_call(item_emb, neigh, w_self_W,
                    w_neigh_W[:HALF], w_neigh_W[HALF:], bias)


# speedup vs baseline: 2.4210x; 1.4850x over previous
import functools

import jax
import jax.numpy as jnp
from jax import lax
from jax.experimental import pallas as pl
from jax.experimental.pallas import tpu as pltpu
from jax.experimental.pallas import tpu_sc as plsc

N_NODES = 10000
N_EDGES = 160000
IN_DIM = 256
HID_DIM = 512

HALF = IN_DIM // 2
CW = 16
NC = 2
NS = 16
EDGES_PER_TILE = N_EDGES // NS
CHUNK = 250
NCHUNK = EDGES_PER_TILE // CHUNK
ROWS_PER_TILE = N_NODES // NS


def _sc_scatter(emb2, ed, ones_h, zf, zc, feat, cnt,
                idx_v, rows_v, ones_v, shf, shc, sem):
    c = lax.axis_index("c")
    s = lax.axis_index("s")
    pltpu.sync_copy(zf, shf.at[pl.ds(s * ROWS_PER_TILE, ROWS_PER_TILE)])
    pltpu.sync_copy(zc, shc.at[pl.ds(s * ROWS_PER_TILE, ROWS_PER_TILE)])
    pltpu.sync_copy(ones_h, ones_v)
    plsc.subcore_barrier()

    def body(j, carry):
        pltpu.sync_copy(ed.at[c, s, j], idx_v)
        pltpu.async_copy(emb2.at[idx_v.at[1]], rows_v, sem).wait()
        pltpu.sync_copy(rows_v, shf.at[idx_v.at[0]], add=True)
        @pl.when(jnp.logical_xor(j < NCHUNK // 2, c == 1))
        def _():
            pltpu.sync_copy(ones_v, shc.at[idx_v.at[0]], add=True)
        return carry

    lax.fori_loop(0, NCHUNK, body, 0)
    plsc.subcore_barrier()
    pltpu.sync_copy(
        shf.at[pl.ds(s * ROWS_PER_TILE, ROWS_PER_TILE)],
        feat.at[c, pl.ds(s * ROWS_PER_TILE, ROWS_PER_TILE)],
    )
    pltpu.sync_copy(
        shc.at[pl.ds(s * ROWS_PER_TILE, ROWS_PER_TILE)],
        cnt.at[c, pl.ds(s * ROWS_PER_TILE, ROWS_PER_TILE)],
    )


_sc_scatter_call = functools.partial(
    pl.kernel,
    out_type=(
        jax.ShapeDtypeStruct((NC, N_NODES, HALF), jnp.float32),
        jax.ShapeDtypeStruct((NC, N_NODES, CW), jnp.float32),
    ),
    mesh=plsc.VectorSubcoreMesh(core_axis_name="c", subcore_axis_name="s"),
    scratch_types=[
        pltpu.VMEM((2, CHUNK), jnp.int32),
        pltpu.VMEM((CHUNK, HALF), jnp.float32),
        pltpu.VMEM((CHUNK, CW), jnp.float32),
        pltpu.VMEM_SHARED((N_NODES, HALF), jnp.float32),
        pltpu.VMEM_SHARED((N_NODES, CW), jnp.float32),
        pltpu.SemaphoreType.DMA,
    ],
    compiler_params=pltpu.CompilerParams(use_tc_tiling_on_sc=False),
)(_sc_scatter)


def _tc_body(x_ref, f_ref, c_ref, ws_ref, wn0_ref, wn1_ref, b_ref, o_ref):
    x = x_ref[...]
    fb = f_ref[...]
    cb = c_ref[...]
    cnt = cb[0, :, :1] + cb[1, :, :1]
    mask = cnt > 0.0
    safe = jnp.where(mask, cnt, 1.0)
    m0 = jnp.where(mask, fb[0] / safe, 0.0)
    m1 = jnp.where(mask, fb[1] / safe, 0.0)
    acc = jnp.dot(x, ws_ref[...], preferred_element_type=jnp.float32)
    acc += jnp.dot(m0, wn0_ref[...], preferred_element_type=jnp.float32)
    acc += jnp.dot(m1, wn1_ref[...], preferred_element_type=jnp.float32)
    acc += b_ref[...]
    acc = jnp.maximum(acc, 0.0)
    nrm = jnp.sqrt(jnp.sum(acc * acc, axis=1, keepdims=True)) + 1e-9
    o_ref[...] = acc / nrm


def _tc_call(x, feat, cnt, ws, wn0, wn1, b):
    R = 1000
    grid = (N_NODES // R,)
    return pl.pallas_call(
        _tc_body,
        grid=grid,
        in_specs=[
            pl.BlockSpec((R, IN_DIM), lambda i: (i, 0)),
            pl.BlockSpec((NC, R, HALF), lambda i: (0, i, 0)),
            pl.BlockSpec((NC, R, CW), lambda i: (0, i, 0)),
            pl.BlockSpec((IN_DIM, HID_DIM), lambda i: (0, 0)),
            pl.BlockSpec((HALF, HID_DIM), lambda i: (0, 0)),
            pl.BlockSpec((HALF, HID_DIM), lambda i: (0, 0)),
            pl.BlockSpec((1, HID_DIM), lambda i: (0, 0)),
        ],
        out_specs=pl.BlockSpec((R, HID_DIM), lambda i: (i, 0)),
        out_shape=jax.ShapeDtypeStruct((N_NODES, HID_DIM), jnp.float32),
    )(x, feat, cnt, ws, wn0, wn1, b)


@jax.jit
def kernel(item_emb, edges, w_self_W, w_self_b, w_neigh_W, w_neigh_b):
    f32 = jnp.float32
    src = edges[:, 0].astype(jnp.int32)
    dst = edges[:, 1].astype(jnp.int32)
    emb2 = item_emb.reshape(2 * N_NODES, HALF)
    srcr = src.reshape(NS, NCHUNK, 1, CHUNK)
    d2 = 2 * dst
    ed = jnp.stack([
        jnp.concatenate([srcr, d2.reshape(NS, NCHUNK, 1, CHUNK)], axis=2),
        jnp.concatenate([srcr, (d2 + 1).reshape(NS, NCHUNK, 1, CHUNK)], axis=2),
    ])
    ones_h = jnp.ones((CHUNK, CW), f32)
    zf = jnp.zeros((ROWS_PER_TILE, HALF), f32)
    zc = jnp.zeros((ROWS_PER_TILE, CW), f32)

    feat, cnt = _sc_scatter_call(emb2, ed, ones_h, zf, zc)

    bias = (w_self_b + w_neigh_b).reshape(1, HID_DIM)
    return _tc_call(item_emb, feat, cnt, w_self_W,
                    w_neigh_W[:HALF], w_neigh_W[HALF:], bias)


# speedup vs baseline: 2.6004x; 1.0741x over previous
import functools

import jax
import jax.numpy as jnp
from jax import lax
from jax.experimental import pallas as pl
from jax.experimental.pallas import tpu as pltpu
from jax.experimental.pallas import tpu_sc as plsc

N_NODES = 10000
N_EDGES = 160000
IN_DIM = 256
HID_DIM = 512

HALF = IN_DIM // 2
CW = 16
NC = 2
NS = 16
EDGES_PER_TILE = N_EDGES // NS
CHUNK = 125
NCHUNK = EDGES_PER_TILE // CHUNK
ROWS_PER_TILE = N_NODES // NS


def _sc_scatter(emb2, ed, ones_h, zf, zc, feat, cnt,
                idx_a, idx_b, rows_a, rows_b, ones_v, shf, shc, sem_a, sem_b):
    c = lax.axis_index("c")
    s = lax.axis_index("s")
    pltpu.sync_copy(zf, shf.at[pl.ds(s * ROWS_PER_TILE, ROWS_PER_TILE)])
    pltpu.sync_copy(zc, shc.at[pl.ds(s * ROWS_PER_TILE, ROWS_PER_TILE)])
    pltpu.sync_copy(ones_h, ones_v)
    plsc.subcore_barrier()

    def half(j, idx_v, rows_v):
        pltpu.sync_copy(rows_v, shf.at[idx_v.at[0]], add=True)
        @pl.when(jnp.logical_xor(j < NCHUNK // 2, c == 1))
        def _():
            pltpu.sync_copy(ones_v, shc.at[idx_v.at[0]], add=True)

    def body(jj, carry):
        a = 2 * jj
        b = a + 1
        pltpu.sync_copy(ed.at[c, s, a], idx_a)
        da = pltpu.async_copy(emb2.at[idx_a.at[1]], rows_a, sem_a)
        pltpu.sync_copy(ed.at[c, s, b], idx_b)
        db = pltpu.async_copy(emb2.at[idx_b.at[1]], rows_b, sem_b)
        da.wait()
        half(a, idx_a, rows_a)
        db.wait()
        half(b, idx_b, rows_b)
        return carry

    lax.fori_loop(0, NCHUNK // 2, body, 0)
    plsc.subcore_barrier()
    pltpu.sync_copy(
        shf.at[pl.ds(s * ROWS_PER_TILE, ROWS_PER_TILE)],
        feat.at[c, pl.ds(s * ROWS_PER_TILE, ROWS_PER_TILE)],
    )
    pltpu.sync_copy(
        shc.at[pl.ds(s * ROWS_PER_TILE, ROWS_PER_TILE)],
        cnt.at[c, pl.ds(s * ROWS_PER_TILE, ROWS_PER_TILE)],
    )


_sc_scatter_call = functools.partial(
    pl.kernel,
    out_type=(
        jax.ShapeDtypeStruct((NC, N_NODES, HALF), jnp.float32),
        jax.ShapeDtypeStruct((NC, N_NODES, CW), jnp.float32),
    ),
    mesh=plsc.VectorSubcoreMesh(core_axis_name="c", subcore_axis_name="s"),
    scratch_types=[
        pltpu.VMEM((2, CHUNK), jnp.int32),
        pltpu.VMEM((2, CHUNK), jnp.int32),
        pltpu.VMEM((CHUNK, HALF), jnp.float32),
        pltpu.VMEM((CHUNK, HALF), jnp.float32),
        pltpu.VMEM((CHUNK, CW), jnp.float32),
        pltpu.VMEM_SHARED((N_NODES, HALF), jnp.float32),
        pltpu.VMEM_SHARED((N_NODES, CW), jnp.float32),
        pltpu.SemaphoreType.DMA,
        pltpu.SemaphoreType.DMA,
    ],
    compiler_params=pltpu.CompilerParams(use_tc_tiling_on_sc=False),
)(_sc_scatter)


def _tc_body(x_ref, f_ref, c_ref, ws_ref, wn0_ref, wn1_ref, b_ref, o_ref):
    x = x_ref[...]
    fb = f_ref[...]
    cb = c_ref[...]
    cnt = cb[0, :, :1] + cb[1, :, :1]
    mask = cnt > 0.0
    safe = jnp.where(mask, cnt, 1.0)
    m0 = jnp.where(mask, fb[0] / safe, 0.0)
    m1 = jnp.where(mask, fb[1] / safe, 0.0)
    acc = jnp.dot(x, ws_ref[...], preferred_element_type=jnp.float32)
    acc += jnp.dot(m0, wn0_ref[...], preferred_element_type=jnp.float32)
    acc += jnp.dot(m1, wn1_ref[...], preferred_element_type=jnp.float32)
    acc += b_ref[...]
    acc = jnp.maximum(acc, 0.0)
    nrm = jnp.sqrt(jnp.sum(acc * acc, axis=1, keepdims=True)) + 1e-9
    o_ref[...] = acc / nrm


def _tc_call(x, feat, cnt, ws, wn0, wn1, b):
    R = 1000
    grid = (N_NODES // R,)
    return pl.pallas_call(
        _tc_body,
        grid=grid,
        in_specs=[
            pl.BlockSpec((R, IN_DIM), lambda i: (i, 0)),
            pl.BlockSpec((NC, R, HALF), lambda i: (0, i, 0)),
            pl.BlockSpec((NC, R, CW), lambda i: (0, i, 0)),
            pl.BlockSpec((IN_DIM, HID_DIM), lambda i: (0, 0)),
            pl.BlockSpec((HALF, HID_DIM), lambda i: (0, 0)),
            pl.BlockSpec((HALF, HID_DIM), lambda i: (0, 0)),
            pl.BlockSpec((1, HID_DIM), lambda i: (0, 0)),
        ],
        out_specs=pl.BlockSpec((R, HID_DIM), lambda i: (i, 0)),
        out_shape=jax.ShapeDtypeStruct((N_NODES, HID_DIM), jnp.float32),
    )(x, feat, cnt, ws, wn0, wn1, b)


@jax.jit
def kernel(item_emb, edges, w_self_W, w_self_b, w_neigh_W, w_neigh_b):
    f32 = jnp.float32
    src = edges[:, 0].astype(jnp.int32)
    dst = edges[:, 1].astype(jnp.int32)
    emb2 = item_emb.reshape(2 * N_NODES, HALF)
    srcr = src.reshape(NS, NCHUNK, 1, CHUNK)
    d2 = 2 * dst
    ed = jnp.stack([
        jnp.concatenate([srcr, d2.reshape(NS, NCHUNK, 1, CHUNK)], axis=2),
        jnp.concatenate([srcr, (d2 + 1).reshape(NS, NCHUNK, 1, CHUNK)], axis=2),
    ])
    ones_h = jnp.ones((CHUNK, CW), f32)
    zf = jnp.zeros((ROWS_PER_TILE, HALF), f32)
    zc = jnp.zeros((ROWS_PER_TILE, CW), f32)

    feat, cnt = _sc_scatter_call(emb2, ed, ones_h, zf, zc)

    bias = (w_self_b + w_neigh_b).reshape(1, HID_DIM)
    return _tc_call(item_emb, feat, cnt, w_self_W,
                    w_neigh_W[:HALF], w_neigh_W[HALF:], bias)


# speedup vs baseline: 3.0361x; 1.1676x over previous
import functools

import jax
import jax.numpy as jnp
from jax import lax
from jax.experimental import pallas as pl
from jax.experimental.pallas import tpu as pltpu
from jax.experimental.pallas import tpu_sc as plsc

N_NODES = 10000
N_EDGES = 160000
IN_DIM = 256
HID_DIM = 512

HALF = IN_DIM // 2
CW = 16
NC = 2
NS = 16
EDGES_PER_TILE = N_EDGES // NS
CHUNK = 125
NCHUNK = EDGES_PER_TILE // CHUNK
ROWS_PER_TILE = N_NODES // NS


def _sc_scatter(emb2, ed, ones_h, zf, zc, feat, cnt,
                idx_a, idx_b, rows_a, rows_b, ones_v, shf, shc,
                sem_ga, sem_gb, sem_sa, sem_sb):
    c = lax.axis_index("c")
    s = lax.axis_index("s")
    pltpu.sync_copy(zf, shf.at[pl.ds(s * ROWS_PER_TILE, ROWS_PER_TILE)])
    pltpu.sync_copy(zc, shc.at[pl.ds(s * ROWS_PER_TILE, ROWS_PER_TILE)])
    pltpu.sync_copy(ones_h, ones_v)
    plsc.subcore_barrier()

    def ones_scatter(j, idx_v):
        @pl.when(jnp.logical_xor(j < NCHUNK // 2, c == 1))
        def _():
            pltpu.sync_copy(ones_v, shc.at[idx_v.at[0]], add=True)

    NJJ = NCHUNK // 2
    pltpu.sync_copy(ed.at[c, s, 0], idx_a)
    pltpu.async_copy(emb2.at[idx_a.at[1]], rows_a, sem_ga)

    def body(jj, carry):
        a = 2 * jj
        b = a + 1
        @pl.when(jj > 0)
        def _():
            pltpu.make_async_copy(emb2.at[pl.ds(0, CHUNK)], rows_b, sem_sb).wait()
        pltpu.sync_copy(ed.at[c, s, b], idx_b)
        pltpu.async_copy(emb2.at[idx_b.at[1]], rows_b, sem_gb)
        pltpu.make_async_copy(emb2.at[pl.ds(0, CHUNK)], rows_a, sem_ga).wait()
        pltpu.async_copy(rows_a, shf.at[idx_a.at[0]], sem_sa, add=True)
        ones_scatter(a, idx_a)
        pltpu.make_async_copy(emb2.at[pl.ds(0, CHUNK)], rows_a, sem_sa).wait()

        @pl.when(jj + 1 < NJJ)
        def _():
            pltpu.sync_copy(ed.at[c, s, a + 2], idx_a)
            pltpu.async_copy(emb2.at[idx_a.at[1]], rows_a, sem_ga)
        pltpu.make_async_copy(emb2.at[pl.ds(0, CHUNK)], rows_b, sem_gb).wait()
        pltpu.async_copy(rows_b, shf.at[idx_b.at[0]], sem_sb, add=True)
        ones_scatter(b, idx_b)
        return carry

    lax.fori_loop(0, NJJ, body, 0)
    pltpu.make_async_copy(emb2.at[pl.ds(0, CHUNK)], rows_b, sem_sb).wait()
    plsc.subcore_barrier()
    pltpu.sync_copy(
        shf.at[pl.ds(s * ROWS_PER_TILE, ROWS_PER_TILE)],
        feat.at[c, pl.ds(s * ROWS_PER_TILE, ROWS_PER_TILE)],
    )
    pltpu.sync_copy(
        shc.at[pl.ds(s * ROWS_PER_TILE, ROWS_PER_TILE)],
        cnt.at[c, pl.ds(s * ROWS_PER_TILE, ROWS_PER_TILE)],
    )


_sc_scatter_call = functools.partial(
    pl.kernel,
    out_type=(
        jax.ShapeDtypeStruct((NC, N_NODES, HALF), jnp.float32),
        jax.ShapeDtypeStruct((NC, N_NODES, CW), jnp.float32),
    ),
    mesh=plsc.VectorSubcoreMesh(core_axis_name="c", subcore_axis_name="s"),
    scratch_types=[
        pltpu.VMEM((2, CHUNK), jnp.int32),
        pltpu.VMEM((2, CHUNK), jnp.int32),
        pltpu.VMEM((CHUNK, HALF), jnp.float32),
        pltpu.VMEM((CHUNK, HALF), jnp.float32),
        pltpu.VMEM((CHUNK, CW), jnp.float32),
        pltpu.VMEM_SHARED((N_NODES, HALF), jnp.float32),
        pltpu.VMEM_SHARED((N_NODES, CW), jnp.float32),
        pltpu.SemaphoreType.DMA,
        pltpu.SemaphoreType.DMA,
        pltpu.SemaphoreType.DMA,
        pltpu.SemaphoreType.DMA,
    ],
    compiler_params=pltpu.CompilerParams(use_tc_tiling_on_sc=False),
)(_sc_scatter)


def _tc_body(x_ref, f_ref, c_ref, ws_ref, wn0_ref, wn1_ref, b_ref, o_ref):
    x = x_ref[...]
    fb = f_ref[...]
    cb = c_ref[...]
    cnt = cb[0, :, :1] + cb[1, :, :1]
    mask = cnt > 0.0
    safe = jnp.where(mask, cnt, 1.0)
    m0 = jnp.where(mask, fb[0] / safe, 0.0)
    m1 = jnp.where(mask, fb[1] / safe, 0.0)
    acc = jnp.dot(x, ws_ref[...], preferred_element_type=jnp.float32)
    acc += jnp.dot(m0, wn0_ref[...], preferred_element_type=jnp.float32)
    acc += jnp.dot(m1, wn1_ref[...], preferred_element_type=jnp.float32)
    acc += b_ref[...]
    acc = jnp.maximum(acc, 0.0)
    nrm = jnp.sqrt(jnp.sum(acc * acc, axis=1, keepdims=True)) + 1e-9
    o_ref[...] = acc / nrm


def _tc_call(x, feat, cnt, ws, wn0, wn1, b):
    R = 1000
    grid = (N_NODES // R,)
    return pl.pallas_call(
        _tc_body,
        grid=grid,
        in_specs=[
            pl.BlockSpec((R, IN_DIM), lambda i: (i, 0)),
            pl.BlockSpec((NC, R, HALF), lambda i: (0, i, 0)),
            pl.BlockSpec((NC, R, CW), lambda i: (0, i, 0)),
            pl.BlockSpec((IN_DIM, HID_DIM), lambda i: (0, 0)),
            pl.BlockSpec((HALF, HID_DIM), lambda i: (0, 0)),
            pl.BlockSpec((HALF, HID_DIM), lambda i: (0, 0)),
            pl.BlockSpec((1, HID_DIM), lambda i: (0, 0)),
        ],
        out_specs=pl.BlockSpec((R, HID_DIM), lambda i: (i, 0)),
        out_shape=jax.ShapeDtypeStruct((N_NODES, HID_DIM), jnp.float32),
    )(x, feat, cnt, ws, wn0, wn1, b)


@jax.jit
def kernel(item_emb, edges, w_self_W, w_self_b, w_neigh_W, w_neigh_b):
    f32 = jnp.float32
    src = edges[:, 0].astype(jnp.int32)
    dst = edges[:, 1].astype(jnp.int32)
    emb2 = item_emb.reshape(2 * N_NODES, HALF)
    srcr = src.reshape(NS, NCHUNK, 1, CHUNK)
    d2 = 2 * dst
    ed = jnp.stack([
        jnp.concatenate([srcr, d2.reshape(NS, NCHUNK, 1, CHUNK)], axis=2),
        jnp.concatenate([srcr, (d2 + 1).reshape(NS, NCHUNK, 1, CHUNK)], axis=2),
    ])
    ones_h = jnp.ones((CHUNK, CW), f32)
    zf = jnp.zeros((ROWS_PER_TILE, HALF), f32)
    zc = jnp.zeros((ROWS_PER_TILE, CW), f32)

    feat, cnt = _sc_scatter_call(emb2, ed, ones_h, zf, zc)

    bias = (w_self_b + w_neigh_b).reshape(1, HID_DIM)
    return _tc_call(item_emb, feat, cnt, w_self_W,
                    w_neigh_W[:HALF], w_neigh_W[HALF:], bias)
